# baseline (speedup 1.0000x reference)
import jax
import jax.numpy as jnp
from jax.experimental import pallas as pl
from jax.experimental.pallas import tpu as pltpu

IMG = 224
FM = 28
Q = FM * FM
KB = 1024
KB2 = 1024


def _phase1_body(patch_t_ref, lib_ref, minval_ref, sstar_ref, star_ref,
                 sidx_ref, b2_ref, a2_ref, cmin_ref, imin_ref):
    i = pl.program_id(0)
    nb = pl.num_programs(0)
    lib = lib_ref[...]
    g = jax.lax.dot_general(lib, patch_t_ref[...],
                            (((1,), (0,)), ((), ())),
                            preferred_element_type=jnp.float32)
    b2 = jnp.sum(lib * lib, axis=1, keepdims=True)
    b2_ref[...] = b2
    c = b2 - 2.0 * g
    bm = jnp.min(c, axis=0, keepdims=True)
    bi = jnp.argmin(c, axis=0).astype(jnp.int32)[None, :] + i * KB

    @pl.when(i == 0)
    def _():
        cmin_ref[...] = bm
        imin_ref[...] = bi

    @pl.when(i > 0)
    def _():
        cur = cmin_ref[...]
        upd = bm < cur
        cmin_ref[...] = jnp.where(upd, bm, cur)
        imin_ref[...] = jnp.where(upd, bi, imin_ref[...])

    @pl.when(i == nb - 1)
    def _():
        pt = patch_t_ref[...]
        a2 = jnp.sum(pt * pt, axis=0, keepdims=True)
        a2_ref[...] = a2
        mv = jnp.sqrt(jnp.maximum(cmin_ref[...] + a2, 0.0))
        minval_ref[...] = mv
        m = jnp.max(mv)
        qio = jax.lax.broadcasted_iota(jnp.int32, (1, Q), 1)
        sidx = jnp.min(jnp.where(mv == m, qio, Q))
        star = jnp.sum(jnp.where(qio == sidx, imin_ref[...], 0))
        sstar_ref[...] = jnp.full((1, 1), m, jnp.float32)
        sidx_ref[...] = jnp.full((1, 1), sidx, jnp.int32)
        star_ref[...] = jnp.full((1, 1), star, jnp.int32)


def _phase2_body(lib_ref, mmt_ref, gs_ref, gt_ref):
    gcol = jax.lax.dot_general(lib_ref[...], mmt_ref[...],
                               (((1,), (0,)), ((), ())),
                               preferred_element_type=jnp.float32)
    gs_ref[...] = gcol[:, 0:1]
    gt_ref[...] = gcol[:, 1:2]


def _phase3_body(sidx_sref, gs_ref, gt_ref, b2_ref, a2_ref, sstar_ref,
                 mval_ref, a_ref, at_ref, s_ref, smap_ref):
    k_tot = gs_ref.shape[1]
    b2 = b2_ref[...]
    qv = b2 - 2.0 * gs_ref[...]
    qio = jax.lax.broadcasted_iota(jnp.int32, (1, Q), 1)
    nt = jnp.sum(jnp.where(qio == sidx_sref[0], a2_ref[...], 0.0))
    dt = jnp.sqrt(jnp.maximum(b2 + nt - 2.0 * gt_ref[...], 0.0))
    io = jax.lax.broadcasted_iota(jnp.int32, (1, k_tot), 1)
    m1 = jnp.min(qv)
    i1 = jnp.min(jnp.where(qv == m1, io, k_tot))
    qv = jnp.where(io == i1, jnp.inf, qv)
    m2 = jnp.min(qv)
    i2 = jnp.min(jnp.where(qv == m2, io, k_tot))
    t2 = jnp.sum(jnp.where(io == i2, dt, 0.0))
    qv = jnp.where(io == i2, jnp.inf, qv)
    m3 = jnp.min(qv)
    i3 = jnp.min(jnp.where(qv == m3, io, k_tot))
    t3 = jnp.sum(jnp.where(io == i3, dt, 0.0))
    dn = jnp.sqrt(jnp.float32(1536))
    sv = sstar_ref[...]
    t2v = jnp.full((1, 1), t2, jnp.float32)
    t3v = jnp.full((1, 1), t3, jnp.float32)
    denom = jnp.exp(t2v / dn) + jnp.exp(t3v / dn)
    s_ref[...] = (1.0 - jnp.exp(sv / dn) / denom) * sv
    tmp = jax.lax.dot_general(a_ref[...], mval_ref[...],
                              (((1,), (0,)), ((), ())),
                              precision=jax.lax.Precision.HIGHEST,
                              preferred_element_type=jnp.float32)
    smap_ref[...] = jax.lax.dot_general(tmp, at_ref[...],
                                        (((1,), (0,)), ((), ())),
                                        precision=jax.lax.Precision.HIGHEST,
                                        preferred_element_type=jnp.float32)


def kernel(patch, patch_lib):
    k_tot, d_feat = patch_lib.shape
    nb = k_tot // KB
    patch_t = patch.T

    minval, sstar, star, sidx, b2col, a2row = pl.pallas_call(
        _phase1_body,
        grid=(nb,),
        in_specs=[
            pl.BlockSpec((d_feat, Q), lambda i: (0, 0)),
            pl.BlockSpec((KB, d_feat), lambda i: (i, 0)),
        ],
        out_specs=[
            pl.BlockSpec((1, Q), lambda i: (0, 0)),
            pl.BlockSpec((1, 1), lambda i: (0, 0)),
            pl.BlockSpec((1, 1), lambda i: (0, 0)),
            pl.BlockSpec((1, 1), lambda i: (0, 0)),
            pl.BlockSpec((KB, 1), lambda i: (i, 0)),
            pl.BlockSpec((1, Q), lambda i: (0, 0)),
        ],
        out_shape=[
            jax.ShapeDtypeStruct((1, Q), jnp.float32),
            jax.ShapeDtypeStruct((1, 1), jnp.float32),
            jax.ShapeDtypeStruct((1, 1), jnp.int32),
            jax.ShapeDtypeStruct((1, 1), jnp.int32),
            jax.ShapeDtypeStruct((k_tot, 1), jnp.float32),
            jax.ShapeDtypeStruct((1, Q), jnp.float32),
        ],
        scratch_shapes=[
            pltpu.VMEM((1, Q), jnp.float32),
            pltpu.VMEM((1, Q), jnp.int32),
        ],
    )(patch_t, patch_lib)

    m_star = jax.lax.dynamic_slice(patch_lib, (star[0, 0], 0), (1, d_feat))
    m_test = jax.lax.dynamic_slice(patch, (sidx[0, 0], 0), (1, d_feat))
    mmt = jnp.concatenate([m_star, m_test], axis=0).T

    gs_col, gt_col = pl.pallas_call(
        _phase2_body,
        grid=(k_tot // KB2,),
        in_specs=[
            pl.BlockSpec((KB2, d_feat), lambda i: (i, 0)),
            pl.BlockSpec((d_feat, 2), lambda i: (0, 0)),
        ],
        out_specs=[
            pl.BlockSpec((KB2, 1), lambda i: (i, 0)),
            pl.BlockSpec((KB2, 1), lambda i: (i, 0)),
        ],
        out_shape=[
            jax.ShapeDtypeStruct((k_tot, 1), jnp.float32),
            jax.ShapeDtypeStruct((k_tot, 1), jnp.float32),
        ],
    )(patch_lib, mmt)

    a_mat = jax.image.resize(jnp.eye(FM, dtype=jnp.float32), (IMG, FM),
                             method="bilinear")
    mval2d = minval.reshape(FM, FM)
    gs_row = gs_col.reshape(1, k_tot)
    gt_row = gt_col.reshape(1, k_tot)
    b2row = b2col.reshape(1, k_tot)

    grid_spec = pltpu.PrefetchScalarGridSpec(
        num_scalar_prefetch=1,
        grid=(1,),
        in_specs=[
            pl.BlockSpec((1, k_tot), lambda i, si: (0, 0)),
            pl.BlockSpec((1, k_tot), lambda i, si: (0, 0)),
            pl.BlockSpec((1, k_tot), lambda i, si: (0, 0)),
            pl.BlockSpec((1, Q), lambda i, si: (0, 0)),
            pl.BlockSpec((1, 1), lambda i, si: (0, 0)),
            pl.BlockSpec((FM, FM), lambda i, si: (0, 0)),
            pl.BlockSpec((IMG, FM), lambda i, si: (0, 0)),
            pl.BlockSpec((FM, IMG), lambda i, si: (0, 0)),
        ],
        out_specs=[
            pl.BlockSpec((1, 1), lambda i, si: (0, 0)),
            pl.BlockSpec((IMG, IMG), lambda i, si: (0, 0)),
        ],
    )

    s_out, smap = pl.pallas_call(
        _phase3_body,
        grid_spec=grid_spec,
        out_shape=[
            jax.ShapeDtypeStruct((1, 1), jnp.float32),
            jax.ShapeDtypeStruct((IMG, IMG), jnp.float32),
        ],
    )(sidx.reshape(1), gs_row, gt_row, b2row, a2row, sstar,
      mval2d, a_mat, a_mat.T)

    return (s_out[0, 0], smap.reshape(1, 1, IMG, IMG))


# speedup vs baseline: 1.0122x; 1.0122x over previous
import jax
import jax.numpy as jnp
from jax.experimental import pallas as pl
from jax.experimental.pallas import tpu as pltpu

IMG = 224
FM = 28
Q = FM * FM
KB = 1024
KB2 = 1024


def _phase1_body(patch_t_ref, lib_ref, minval_ref, sstar_ref, star_ref,
                 sidx_ref, b2_ref, a2_ref, cmin_ref, imin_ref):
    i = pl.program_id(0)
    nb = pl.num_programs(0)
    lib = lib_ref[...]
    g = jax.lax.dot_general(lib, patch_t_ref[...],
                            (((1,), (0,)), ((), ())),
                            preferred_element_type=jnp.float32)
    b2 = jnp.sum(lib * lib, axis=1, keepdims=True)
    b2_ref[...] = b2
    c = b2 - 2.0 * g
    bm = jnp.min(c, axis=0, keepdims=True)
    bi = jnp.argmin(c, axis=0).astype(jnp.int32)[None, :] + i * KB

    @pl.when(i == 0)
    def _():
        cmin_ref[...] = bm
        imin_ref[...] = bi

    @pl.when(i > 0)
    def _():
        cur = cmin_ref[...]
        upd = bm < cur
        cmin_ref[...] = jnp.where(upd, bm, cur)
        imin_ref[...] = jnp.where(upd, bi, imin_ref[...])

    @pl.when(i == nb - 1)
    def _():
        pt = patch_t_ref[...]
        a2 = jnp.sum(pt * pt, axis=0, keepdims=True)
        a2_ref[...] = a2
        mv = jnp.sqrt(jnp.maximum(cmin_ref[...] + a2, 0.0))
        minval_ref[...] = mv
        m = jnp.max(mv)
        qio = jax.lax.broadcasted_iota(jnp.int32, (1, Q), 1)
        sidx = jnp.min(jnp.where(mv == m, qio, Q))
        star = jnp.sum(jnp.where(qio == sidx, imin_ref[...], 0))
        sstar_ref[...] = jnp.full((1, 1), m, jnp.float32)
        sidx_ref[...] = jnp.full((1, 1), sidx, jnp.int32)
        star_ref[...] = jnp.full((1, 1), star, jnp.int32)


def _phase2_body(lib_ref, mmt_ref, gcol_ref):
    gcol_ref[...] = jax.lax.dot_general(lib_ref[...], mmt_ref[...],
                                        (((1,), (0,)), ((), ())),
                                        preferred_element_type=jnp.float32)


def _phase3_body(sidx_sref, gs_ref, gt_ref, b2_ref, a2_ref, sstar_ref,
                 mval_ref, a_ref, at_ref, s_ref, smap_ref):
    k_tot = gs_ref.shape[1]
    b2 = b2_ref[...]
    qv = b2 - 2.0 * gs_ref[...]
    qio = jax.lax.broadcasted_iota(jnp.int32, (1, Q), 1)
    nt = jnp.sum(jnp.where(qio == sidx_sref[0], a2_ref[...], 0.0))
    dt = jnp.sqrt(jnp.maximum(b2 + nt - 2.0 * gt_ref[...], 0.0))
    io = jax.lax.broadcasted_iota(jnp.int32, (1, k_tot), 1)
    m1 = jnp.min(qv)
    i1 = jnp.min(jnp.where(qv == m1, io, k_tot))
    qv = jnp.where(io == i1, jnp.inf, qv)
    m2 = jnp.min(qv)
    i2 = jnp.min(jnp.where(qv == m2, io, k_tot))
    t2 = jnp.sum(jnp.where(io == i2, dt, 0.0))
    qv = jnp.where(io == i2, jnp.inf, qv)
    m3 = jnp.min(qv)
    i3 = jnp.min(jnp.where(qv == m3, io, k_tot))
    t3 = jnp.sum(jnp.where(io == i3, dt, 0.0))
    dn = jnp.sqrt(jnp.float32(1536))
    sv = sstar_ref[...]
    t2v = jnp.full((1, 1), t2, jnp.float32)
    t3v = jnp.full((1, 1), t3, jnp.float32)
    denom = jnp.exp(t2v / dn) + jnp.exp(t3v / dn)
    s_ref[...] = (1.0 - jnp.exp(sv / dn) / denom) * sv
    tmp = jax.lax.dot_general(a_ref[...], mval_ref[...],
                              (((1,), (0,)), ((), ())),
                              precision=jax.lax.Precision.HIGHEST,
                              preferred_element_type=jnp.float32)
    smap_ref[...] = jax.lax.dot_general(tmp, at_ref[...],
                                        (((1,), (0,)), ((), ())),
                                        precision=jax.lax.Precision.HIGHEST,
                                        preferred_element_type=jnp.float32)


def kernel(patch, patch_lib):
    k_tot, d_feat = patch_lib.shape
    nb = k_tot // KB
    patch_t = patch.T

    minval, sstar, star, sidx, b2col, a2row = pl.pallas_call(
        _phase1_body,
        grid=(nb,),
        in_specs=[
            pl.BlockSpec((d_feat, Q), lambda i: (0, 0)),
            pl.BlockSpec((KB, d_feat), lambda i: (i, 0)),
        ],
        out_specs=[
            pl.BlockSpec((1, Q), lambda i: (0, 0)),
            pl.BlockSpec((1, 1), lambda i: (0, 0)),
            pl.BlockSpec((1, 1), lambda i: (0, 0)),
            pl.BlockSpec((1, 1), lambda i: (0, 0)),
            pl.BlockSpec((KB, 1), lambda i: (i, 0)),
            pl.BlockSpec((1, Q), lambda i: (0, 0)),
        ],
        out_shape=[
            jax.ShapeDtypeStruct((1, Q), jnp.float32),
            jax.ShapeDtypeStruct((1, 1), jnp.float32),
            jax.ShapeDtypeStruct((1, 1), jnp.int32),
            jax.ShapeDtypeStruct((1, 1), jnp.int32),
            jax.ShapeDtypeStruct((k_tot, 1), jnp.float32),
            jax.ShapeDtypeStruct((1, Q), jnp.float32),
        ],
        scratch_shapes=[
            pltpu.VMEM((1, Q), jnp.float32),
            pltpu.VMEM((1, Q), jnp.int32),
        ],
    )(patch_t, patch_lib)

    m_star = jax.lax.dynamic_slice(patch_lib, (star[0, 0], 0), (1, d_feat))
    m_test = jax.lax.dynamic_slice(patch, (sidx[0, 0], 0), (1, d_feat))
    mmt = jnp.concatenate([m_star, m_test], axis=0).T

    gcols = pl.pallas_call(
        _phase2_body,
        grid=(k_tot // KB2,),
        in_specs=[
            pl.BlockSpec((KB2, d_feat), lambda i: (i, 0)),
            pl.BlockSpec((d_feat, 2), lambda i: (0, 0)),
        ],
        out_specs=pl.BlockSpec((KB2, 2), lambda i: (i, 0)),
        out_shape=jax.ShapeDtypeStruct((k_tot, 2), jnp.float32),
    )(patch_lib, mmt)

    a_mat = jax.image.resize(jnp.eye(FM, dtype=jnp.float32), (IMG, FM),
                             method="bilinear")
    mval2d = minval.reshape(FM, FM)
    gs_row = gcols[:, 0].reshape(1, k_tot)
    gt_row = gcols[:, 1].reshape(1, k_tot)
    b2row = b2col.reshape(1, k_tot)

    grid_spec = pltpu.PrefetchScalarGridSpec(
        num_scalar_prefetch=1,
        grid=(1,),
        in_specs=[
            pl.BlockSpec((1, k_tot), lambda i, si: (0, 0)),
            pl.BlockSpec((1, k_tot), lambda i, si: (0, 0)),
            pl.BlockSpec((1, k_tot), lambda i, si: (0, 0)),
            pl.BlockSpec((1, Q), lambda i, si: (0, 0)),
            pl.BlockSpec((1, 1), lambda i, si: (0, 0)),
            pl.BlockSpec((FM, FM), lambda i, si: (0, 0)),
            pl.BlockSpec((IMG, FM), lambda i, si: (0, 0)),
            pl.BlockSpec((FM, IMG), lambda i, si: (0, 0)),
        ],
        out_specs=[
            pl.BlockSpec((1, 1), lambda i, si: (0, 0)),
            pl.BlockSpec((IMG, IMG), lambda i, si: (0, 0)),
        ],
    )

    s_out, smap = pl.pallas_call(
        _phase3_body,
        grid_spec=grid_spec,
        out_shape=[
            jax.ShapeDtypeStruct((1, 1), jnp.float32),
            jax.ShapeDtypeStruct((IMG, IMG), jnp.float32),
        ],
    )(sidx.reshape(1), gs_row, gt_row, b2row, a2row, sstar,
      mval2d, a_mat, a_mat.T)

    return (s_out[0, 0], smap.reshape(1, 1, IMG, IMG))
